# baseline (device time: 18701 ns/iter reference)
import jax
import jax.numpy as jnp
from jax import lax
from jax.experimental import pallas as pl
from jax.experimental.pallas import tpu as pltpu

N_DEV = 8


def kernel(x):
    m, n = x.shape
    C = 128
    NB = m // C

    def body(x_hbm, o_hbm, xv, ov, tot_ref, recv_ref,
             in_sems, out_sems, send_sems, recv_sems, ack_sem):
        my = lax.axis_index("i")

        def in_copy(c):
            return pltpu.make_async_copy(
                x_hbm.at[pl.ds(c * C, C), :],
                xv.at[pl.ds(c * C, C), :],
                in_sems.at[c],
            )

        for c in range(NB):
            in_copy(c).start()

        tsum = jnp.zeros((1, n), jnp.float32)
        for c in range(NB):
            in_copy(c).wait()
            tsum = tsum + jnp.sum(
                xv[pl.ds(c * C, C), :], axis=0, keepdims=True
            )
        tot_ref[...] = tsum

        for j in range(N_DEV):
            for k in range(j + 1, N_DEV):
                @pl.when(my == j)
                def _(j=j, k=k):
                    pltpu.make_async_remote_copy(
                        src_ref=tot_ref,
                        dst_ref=recv_ref.at[j],
                        send_sem=send_sems.at[k],
                        recv_sem=recv_sems.at[j],
                        device_id=(k,),
                        device_id_type=pl.DeviceIdType.MESH,
                    ).start()

        row = lax.broadcasted_iota(jnp.int32, (C, C), 0)
        col = lax.broadcasted_iota(jnp.int32, (C, C), 1)
        tri = (row >= col).astype(jnp.float32)
        for c in range(NB):
            blk = xv[pl.ds(c * C, C), :]
            xv[pl.ds(c * C, C), :] = jnp.dot(
                tri, blk, preferred_element_type=jnp.float32
            )

        for k in range(N_DEV):
            @pl.when(my == k)
            def _(k=k):
                for j in range(k):
                    pltpu.make_async_remote_copy(
                        src_ref=tot_ref,
                        dst_ref=recv_ref.at[j],
                        send_sem=send_sems.at[k],
                        recv_sem=recv_sems.at[j],
                        device_id=(j,),
                        device_id_type=pl.DeviceIdType.MESH,
                    ).wait_recv()
                    pl.semaphore_signal(
                        ack_sem, inc=1,
                        device_id=(j,),
                        device_id_type=pl.DeviceIdType.MESH,
                    )

        off = jnp.zeros((1, n), jnp.float32)
        for j in range(N_DEV - 1):
            off = off + jnp.where(j < my, recv_ref[j], 0.0)

        def out_copy(c, s):
            return pltpu.make_async_copy(
                ov.at[s], o_hbm.at[pl.ds(c * C, C), :], out_sems.at[s]
            )

        carry = off
        for c in range(NB):
            s = c % 2
            if c >= 2:
                out_copy(c - 2, s).wait()
            pref = xv[pl.ds(c * C, C), :]
            ov[s, :, :] = pref + carry
            carry = carry + xv[pl.ds(c * C + C - 1, 1), :]
            out_copy(c, s).start()
        out_copy(NB - 2, (NB - 2) % 2).wait()
        out_copy(NB - 1, (NB - 1) % 2).wait()

        for j in range(N_DEV):
            @pl.when(my == j)
            def _(j=j):
                for k in range(j + 1, N_DEV):
                    pltpu.make_async_remote_copy(
                        src_ref=tot_ref,
                        dst_ref=recv_ref.at[j],
                        send_sem=send_sems.at[k],
                        recv_sem=recv_sems.at[j],
                        device_id=(k,),
                        device_id_type=pl.DeviceIdType.MESH,
                    ).wait_send()
                for _ in range(j + 1, N_DEV):
                    pl.semaphore_wait(ack_sem, 1)

    return pl.pallas_call(
        body,
        out_shape=jax.ShapeDtypeStruct((m, n), x.dtype),
        in_specs=[pl.BlockSpec(memory_space=pl.ANY)],
        out_specs=pl.BlockSpec(memory_space=pl.ANY),
        scratch_shapes=[
            pltpu.VMEM((m, n), x.dtype),
            pltpu.VMEM((2, C, n), x.dtype),
            pltpu.VMEM((1, n), x.dtype),
            pltpu.VMEM((N_DEV, 1, n), x.dtype),
            pltpu.SemaphoreType.DMA((NB,)),
            pltpu.SemaphoreType.DMA((2,)),
            pltpu.SemaphoreType.DMA((N_DEV,)),
            pltpu.SemaphoreType.DMA((N_DEV,)),
            pltpu.SemaphoreType.REGULAR,
        ],
    )(x)


# device time: 16146 ns/iter; 1.1582x vs baseline; 1.1582x over previous
import jax
import jax.numpy as jnp
from jax import lax
from jax.experimental import pallas as pl
from jax.experimental.pallas import tpu as pltpu

N_DEV = 8
NPRE = 2


def kernel(x):
    m, n = x.shape
    C = 256
    NB = m // C

    def body(x_hbm, o_hbm, xv, pv, ov, tot_ref, recv_ref,
             in_sems, out_sems, send_sems, recv_sems, ack_sem):
        my = lax.axis_index("i")

        barrier_sem = pltpu.get_barrier_semaphore()
        for p in range(N_DEV):
            @pl.when(my != p)
            def _(p=p):
                pl.semaphore_signal(
                    barrier_sem, inc=1,
                    device_id=(p,), device_id_type=pl.DeviceIdType.MESH)

        def in_copy(c):
            return pltpu.make_async_copy(
                x_hbm.at[pl.ds(c * C, C), :], xv.at[pl.ds(c * C, C), :],
                in_sems.at[c])

        for c in range(NB):
            in_copy(c).start()
        for c in range(NB):
            in_copy(c).wait()

        ones = jnp.ones((1, m), jnp.float32)
        tot_ref[...] = jnp.dot(ones, xv[...], preferred_element_type=jnp.float32)

        pl.semaphore_wait(barrier_sem, N_DEV - 1)

        def rdma(j, k):
            return pltpu.make_async_remote_copy(
                src_ref=tot_ref, dst_ref=recv_ref.at[j],
                send_sem=send_sems.at[k], recv_sem=recv_sems.at[j],
                device_id=(k,), device_id_type=pl.DeviceIdType.MESH)

        for j in range(N_DEV):
            for k in range(j + 1, N_DEV):
                @pl.when(my == j)
                def _(j=j, k=k):
                    rdma(j, k).start()

        row = lax.broadcasted_iota(jnp.int32, (C, C), 0)
        col = lax.broadcasted_iota(jnp.int32, (C, C), 1)
        tri = (row >= col).astype(jnp.float32)

        for c in range(NPRE):
            pv[pl.ds(c * C, C), :] = jnp.dot(
                tri, xv[pl.ds(c * C, C), :], preferred_element_type=jnp.float32)

        for k in range(N_DEV):
            @pl.when(my == k)
            def _(k=k):
                for j in range(k):
                    rdma(j, k).wait_recv()
                    pl.semaphore_signal(
                        ack_sem, inc=1,
                        device_id=(j,), device_id_type=pl.DeviceIdType.MESH)

        off = jnp.zeros((1, n), jnp.float32)
        for j in range(N_DEV - 1):
            off = off + jnp.where(j < my, recv_ref[j], 0.0)

        def out_copy(c, s):
            return pltpu.make_async_copy(
                ov.at[s], o_hbm.at[pl.ds(c * C, C), :], out_sems.at[s])

        carry = off
        for c in range(NB):
            s = c % 2
            if c >= 2:
                out_copy(c - 2, s).wait()
            if c < NPRE:
                pref = pv[pl.ds(c * C, C), :]
            else:
                pref = jnp.dot(
                    tri, xv[pl.ds(c * C, C), :],
                    preferred_element_type=jnp.float32)
            ov[s, :, :] = pref + carry
            out_copy(c, s).start()
            carry = ov[s, C - 1 :, :]
        out_copy(NB - 2, (NB - 2) % 2).wait()
        out_copy(NB - 1, (NB - 1) % 2).wait()

        for j in range(N_DEV):
            @pl.when(my == j)
            def _(j=j):
                for k in range(j + 1, N_DEV):
                    rdma(j, k).wait_send()
                for _ in range(j + 1, N_DEV):
                    pl.semaphore_wait(ack_sem, 1)

    return pl.pallas_call(
        body,
        out_shape=jax.ShapeDtypeStruct((m, n), x.dtype),
        in_specs=[pl.BlockSpec(memory_space=pl.ANY)],
        out_specs=pl.BlockSpec(memory_space=pl.ANY),
        scratch_shapes=[
            pltpu.VMEM((m, n), x.dtype),
            pltpu.VMEM((NPRE * C, n), x.dtype),
            pltpu.VMEM((2, C, n), x.dtype),
            pltpu.VMEM((1, n), x.dtype),
            pltpu.VMEM((N_DEV, 1, n), x.dtype),
            pltpu.SemaphoreType.DMA((NB,)),
            pltpu.SemaphoreType.DMA((2,)),
            pltpu.SemaphoreType.DMA((N_DEV,)),
            pltpu.SemaphoreType.DMA((N_DEV,)),
            pltpu.SemaphoreType.REGULAR,
        ],
        compiler_params=pltpu.CompilerParams(collective_id=0),
    )(x)
